# baseline (speedup 1.0000x reference)
import functools

import jax
import jax.numpy as jnp
from jax import lax
from jax.experimental import pallas as pl
from jax.experimental.pallas import tpu as pltpu
from jax.experimental.pallas import tpu_sc as plsc

N = 10000
H = 128
MF = 20
E = 320000
LBL = 65536
NSUB = 16

CH = 100
ER = E // CH
CPT = ER // NSUB
BLK = 8
RING = 4
NBLK = CPT // BLK
NT = 632
NTL = N - 15 * NT

CCH = 128
IR = LBL // CCH
NW = 2 * NSUB
EPW = LBL // NW
NCH = EPW // CCH

_MESH = plsc.VectorSubcoreMesh(core_axis_name="c", subcore_axis_name="s")


def _seg_sum(table_m, table_u, esrc, edst, z2d, z1d, with_counts):
    out_type = [jax.ShapeDtypeStruct((N, H), jnp.float32),
                jax.ShapeDtypeStruct((N, H), jnp.float32)]
    scratch = [
        pltpu.VMEM((RING * BLK, CH), jnp.int32),
        pltpu.VMEM((RING * BLK, CH), jnp.int32),
        pltpu.VMEM((CH, H), jnp.float32),
        pltpu.VMEM((CH, H), jnp.float32),
        pltpu.VMEM((CH, H), jnp.float32),
        pltpu.VMEM_SHARED((N, H), jnp.float32),
        pltpu.SemaphoreType.DMA,
        pltpu.SemaphoreType.DMA,
        pltpu.SemaphoreType.DMA,
        pltpu.SemaphoreType.DMA,
        pltpu.SemaphoreType.DMA,
        pltpu.SemaphoreType.DMA,
        pltpu.SemaphoreType.DMA,
        pltpu.SemaphoreType.DMA,
        pltpu.SemaphoreType.DMA,
        pltpu.SemaphoreType.DMA,
    ]
    if with_counts:
        out_type += [jax.ShapeDtypeStruct((N,), jnp.float32),
                     jax.ShapeDtypeStruct((N,), jnp.float32)]
        scratch += [pltpu.VMEM((128,), jnp.float32),
                    pltpu.VMEM_SHARED((N,), jnp.float32),
                    pltpu.SemaphoreType.DMA]

    def body(*refs):
        if with_counts:
            (tm, tu, es, ed, zz2, zz1,
             outm, outu, cntm, cntu,
             gring, sring, b0, b1, b2, acc,
             g0, g1, g2, s0, s1, s2, i0, i1, i2, i3,
             ones_v, cacc, csem) = refs
        else:
            (tm, tu, es, ed, zz2, zz1,
             outm, outu,
             gring, sring, b0, b1, b2, acc,
             g0, g1, g2, s0, s1, s2, i0, i1, i2, i3) = refs
        bufs = (b0, b1, b2)
        gsem = (g0, g1, g2)
        ssem = (s0, s1, s2)
        isem = (i0, i1, i2, i3)
        cid = lax.axis_index("c")
        sid = lax.axis_index("s")

        @pl.when(sid < 15)
        def _():
            pltpu.sync_copy(zz2, acc.at[pl.ds(sid * NT, NT)])

        @pl.when(sid == 15)
        def _():
            pltpu.sync_copy(zz2.at[pl.ds(0, NTL)], acc.at[pl.ds(15 * NT, NTL)])
        if with_counts:
            @pl.when(sid == 0)
            def _():
                pltpu.sync_copy(zz1, cacc)
            for k in range(8):
                ones_v[pl.ds(k * 16, 16)] = jnp.ones((16,), jnp.float32)

        plsc.subcore_barrier()

        def run_pass(table, g_hbm, s_hbm):
            row0 = sid * CPT

            def idx_slice(hbm, blk):
                return hbm.at[pl.ds(row0 + blk * BLK, BLK)]

            def ring_slice(ring, blk):
                return ring.at[pl.ds((blk % RING) * BLK, BLK)]

            def idx_fire(blk):
                sem = isem[blk % RING]
                pltpu.async_copy(idx_slice(g_hbm, blk), ring_slice(gring, blk), sem)
                pltpu.async_copy(idx_slice(s_hbm, blk), ring_slice(sring, blk), sem)

            def idx_wait(blk):
                sem = isem[blk % RING]
                pltpu.make_async_copy(idx_slice(g_hbm, blk),
                                      ring_slice(gring, blk), sem).wait()
                pltpu.make_async_copy(idx_slice(s_hbm, blk),
                                      ring_slice(sring, blk), sem).wait()

            def g_desc(c):
                slot = c % 3
                return pltpu.make_async_copy(
                    table.at[gring.at[c % (RING * BLK)]], bufs[slot], gsem[slot])

            def s_desc(c):
                slot = c % 3
                return pltpu.make_async_copy(
                    bufs[slot], acc.at[sring.at[c % (RING * BLK)]], ssem[slot])

            def c_desc(c):
                return pltpu.make_async_copy(
                    ones_v.at[pl.ds(0, CH)],
                    cacc.at[sring.at[c % (RING * BLK)]], csem)

            idx_fire(0)
            idx_fire(1)
            idx_waited = set()
            idx_wait(0)
            idx_waited.add(0)
            g_desc(0).start()
            if 1 // BLK not in idx_waited:
                idx_wait(1 // BLK)
                idx_waited.add(1 // BLK)
            g_desc(1).start()

            for c in range(CPT):
                if c % BLK == 0 and c // BLK + 2 < NBLK:
                    idx_fire(c // BLK + 2)
                g_desc(c).wait()
                s_desc(c).start(add=True)
                if with_counts:
                    if c >= 1:
                        c_desc(c - 1).wait()
                    c_desc(c).start(add=True)
                c2 = c + 2
                if c2 < CPT:
                    if c2 >= 3:
                        s_desc(c2 - 3).wait()
                    if c2 // BLK not in idx_waited:
                        idx_wait(c2 // BLK)
                        idx_waited.add(c2 // BLK)
                    g_desc(c2).start()

            for c in (CPT - 3, CPT - 2, CPT - 1):
                s_desc(c).wait()
            if with_counts:
                c_desc(CPT - 1).wait()

        @pl.when(cid == 0)
        def _():
            run_pass(tm, es, ed)

        @pl.when(cid == 1)
        def _():
            run_pass(tu, ed, es)

        plsc.subcore_barrier()

        def writeback(out, cnt_out, cacc_ref):
            @pl.when(sid < 15)
            def _():
                pltpu.sync_copy(acc.at[pl.ds(sid * NT, NT)],
                                out.at[pl.ds(sid * NT, NT)])

            @pl.when(sid == 15)
            def _():
                pltpu.sync_copy(acc.at[pl.ds(15 * NT, NTL)],
                                out.at[pl.ds(15 * NT, NTL)])
            if with_counts:
                @pl.when(sid == 0)
                def _():
                    pltpu.sync_copy(cacc_ref, cnt_out)

        @pl.when(cid == 0)
        def _():
            writeback(outm, cntm if with_counts else None,
                      cacc if with_counts else None)

        @pl.when(cid == 1)
        def _():
            writeback(outu, cntu if with_counts else None,
                      cacc if with_counts else None)

    k = pl.kernel(body, out_type=tuple(out_type), mesh=_MESH,
                  scratch_types=scratch)
    return k(table_m, table_u, esrc, edst, z2d, z1d)


def _movie_input(movie_x, w, b, emb):
    def body(mx, w_, b_, e_, o):
        o[...] = (jnp.dot(mx[...], w_[...], preferred_element_type=jnp.float32)
                  + b_[...] + e_[...])

    return pl.pallas_call(
        body,
        grid=(10,),
        in_specs=[pl.BlockSpec((1000, MF), lambda i: (i, 0)),
                  pl.BlockSpec((MF, H), lambda i: (0, 0)),
                  pl.BlockSpec((1, H), lambda i: (0, 0)),
                  pl.BlockSpec((1000, H), lambda i: (i, 0))],
        out_specs=pl.BlockSpec((1000, H), lambda i: (i, 0)),
        out_shape=jax.ShapeDtypeStruct((N, H), jnp.float32),
    )(movie_x, w, b, emb)


def _sage_linear(aggm, cbm, xm, wlm, blm, wrm,
                 aggu, cbu, xu, wlu, blu, wru, relu):
    def body(am, cm, xm_, wl1, bl1, wr1, au, cu, xu_, wl2, bl2, wr2, om, ou):
        mm = am[...] / jnp.maximum(cm[...], 1.0)
        hm = (jnp.dot(mm, wl1[...], preferred_element_type=jnp.float32)
              + bl1[...]
              + jnp.dot(xm_[...], wr1[...], preferred_element_type=jnp.float32))
        mu = au[...] / jnp.maximum(cu[...], 1.0)
        hu = (jnp.dot(mu, wl2[...], preferred_element_type=jnp.float32)
              + bl2[...]
              + jnp.dot(xu_[...], wr2[...], preferred_element_type=jnp.float32))
        if relu:
            hm = jnp.maximum(hm, 0.0)
            hu = jnp.maximum(hu, 0.0)
        om[...] = hm
        ou[...] = hu

    row = pl.BlockSpec((1000, H), lambda i: (i, 0))
    cnt = pl.BlockSpec((1000, 1), lambda i: (i, 0))
    wspec = pl.BlockSpec((H, H), lambda i: (0, 0))
    bspec = pl.BlockSpec((1, H), lambda i: (0, 0))
    return pl.pallas_call(
        body,
        grid=(10,),
        in_specs=[row, cnt, row, wspec, bspec, wspec,
                  row, cnt, row, wspec, bspec, wspec],
        out_specs=[row, row],
        out_shape=[jax.ShapeDtypeStruct((N, H), jnp.float32),
                   jax.ShapeDtypeStruct((N, H), jnp.float32)],
    )(aggm, cbm, xm, wlm, blm, wrm, aggu, cbu, xu, wlu, blu, wru)


def _edge_products(xu2, xm2, ls, ld):
    @functools.partial(
        pl.kernel,
        out_type=jax.ShapeDtypeStruct((LBL, H), jnp.float32),
        mesh=_MESH,
        scratch_types=[
            pltpu.VMEM((NCH, CCH), jnp.int32),
            pltpu.VMEM((NCH, CCH), jnp.int32),
            pltpu.VMEM((CCH, H), jnp.float32),
            pltpu.VMEM((CCH, H), jnp.float32),
            pltpu.VMEM((CCH, H), jnp.float32),
            pltpu.VMEM((CCH, H), jnp.float32),
            pltpu.SemaphoreType.DMA,
            pltpu.SemaphoreType.DMA,
            pltpu.SemaphoreType.DMA,
            pltpu.SemaphoreType.DMA,
        ],
    )
    def k(xu, xm, lsr, ldr, prod, iu, im, ub0, mb0, ub1, mb1,
          sema, semb, semo0, semo1):
        cid = lax.axis_index("c")
        sid = lax.axis_index("s")
        w = sid * 2 + cid
        base = w * EPW
        pltpu.sync_copy(lsr.at[pl.ds(w * NCH, NCH)], iu)
        pltpu.sync_copy(ldr.at[pl.ds(w * NCH, NCH)], im)

        def compute(ub, mb):
            def row_body(r, carry):
                for k in range(H // 16):
                    sl = pl.ds(k * 16, 16)
                    ub[r, sl] = ub[r, sl] * mb[r, sl]
                return carry

            lax.fori_loop(0, CCH, row_body, 0)

        def out_slice(c):
            return prod.at[pl.ds(base + c * CCH, CCH)]

        def gather(c, ub, mb, sem):
            pltpu.async_copy(xu.at[iu.at[c]], ub, sem)
            pltpu.async_copy(xm.at[im.at[c]], mb, sem)

        def gwait(c, ub, mb, sem):
            pltpu.make_async_copy(xu.at[iu.at[c]], ub, sem).wait()
            pltpu.make_async_copy(xm.at[im.at[c]], mb, sem).wait()

        gather(0, ub0, mb0, sema)

        def pair(c2, carry):
            c = c2 * 2
            gwait(c, ub0, mb0, sema)

            @pl.when(c >= 1)
            def _():
                pltpu.make_async_copy(ub1, out_slice(c - 1), semo1).wait()

            gather(c + 1, ub1, mb1, semb)
            compute(ub0, mb0)
            pltpu.async_copy(ub0, out_slice(c), semo0)
            gwait(c + 1, ub1, mb1, semb)

            @pl.when(c + 2 < NCH)
            def _():
                pltpu.make_async_copy(ub0, out_slice(c), semo0).wait()
                gather(c + 2, ub0, mb0, sema)

            compute(ub1, mb1)
            pltpu.async_copy(ub1, out_slice(c + 1), semo1)
            return carry

        lax.fori_loop(0, NCH // 2, pair, 0)
        pltpu.make_async_copy(ub0, out_slice(NCH - 2), semo0).wait()
        pltpu.make_async_copy(ub1, out_slice(NCH - 1), semo1).wait()

    return k(xu2, xm2, ls, ld)


def _row_sum(prod):
    def body(p, o):
        ones = jnp.ones((H, 1), jnp.float32)
        o[...] = jnp.dot(p[...], ones, preferred_element_type=jnp.float32)

    out = pl.pallas_call(
        body,
        grid=(16,),
        in_specs=[pl.BlockSpec((LBL // 16, H), lambda i: (i, 0))],
        out_specs=pl.BlockSpec((LBL // 16, 1), lambda i: (i, 0)),
        out_shape=jax.ShapeDtypeStruct((LBL, 1), jnp.float32),
    )(prod)
    return out.reshape(LBL)


def kernel(user_node_id, movie_node_id, movie_x, edge_index, edge_label_index,
           user_emb, movie_emb, movie_lin_w, movie_lin_b,
           c1m_wl, c1m_bl, c1m_wr, c1u_wl, c1u_bl, c1u_wr,
           c2m_wl, c2m_bl, c2m_wr, c2u_wl, c2u_bl, c2u_wr):
    x_user = user_emb
    x_movie = _movie_input(movie_x, movie_lin_w,
                           movie_lin_b.reshape(1, H), movie_emb)

    esrc = edge_index[0].reshape(ER, CH)
    edst = edge_index[1].reshape(ER, CH)
    z2d = jnp.zeros((NT, H), jnp.float32)
    z1d = jnp.zeros((N,), jnp.float32)

    agg1m, agg1u, cntm, cntu = _seg_sum(x_user, x_movie, esrc, edst,
                                        z2d, z1d, with_counts=True)
    cbm = cntm.reshape(N, 1)
    cbu = cntu.reshape(N, 1)
    xm1, xu1 = _sage_linear(agg1m, cbm, x_movie,
                            c1m_wl, c1m_bl.reshape(1, H), c1m_wr,
                            agg1u, cbu, x_user,
                            c1u_wl, c1u_bl.reshape(1, H), c1u_wr, relu=True)

    agg2m, agg2u = _seg_sum(xu1, xm1, esrc, edst, z2d, z1d,
                            with_counts=False)
    xm2, xu2 = _sage_linear(agg2m, cbm, xm1,
                            c2m_wl, c2m_bl.reshape(1, H), c2m_wr,
                            agg2u, cbu, xu1,
                            c2u_wl, c2u_bl.reshape(1, H), c2u_wr, relu=False)

    ls = edge_label_index[0].reshape(IR, CCH)
    ld = edge_label_index[1].reshape(IR, CCH)
    return _row_sum(_edge_products(xu2, xm2, ls, ld))


# speedup vs baseline: 1.0214x; 1.0214x over previous
import functools

import jax
import jax.numpy as jnp
from jax import lax
from jax.experimental import pallas as pl
from jax.experimental.pallas import tpu as pltpu
from jax.experimental.pallas import tpu_sc as plsc

N = 10000
H = 128
MF = 20
E = 320000
LBL = 65536
NSUB = 16

CH = 100
ER = E // CH
CPT = ER // NSUB
BLK = 8
RING = 4
NBLK = CPT // BLK
NT = 632
NTL = N - 15 * NT

CCH = 128
IR = LBL // CCH
NW = 2 * NSUB
EPW = LBL // NW
NCH = EPW // CCH

_MESH = plsc.VectorSubcoreMesh(core_axis_name="c", subcore_axis_name="s")


def _seg_sum(table_m, table_u, esrc, edst, z2d, z1d, with_counts):
    out_type = [jax.ShapeDtypeStruct((N, H), jnp.float32),
                jax.ShapeDtypeStruct((N, H), jnp.float32)]
    scratch = [
        pltpu.VMEM((RING * BLK, CH), jnp.int32),
        pltpu.VMEM((RING * BLK, CH), jnp.int32),
        pltpu.VMEM((CH, H), jnp.float32),
        pltpu.VMEM((CH, H), jnp.float32),
        pltpu.VMEM((CH, H), jnp.float32),
        pltpu.VMEM_SHARED((N, H), jnp.float32),
        pltpu.SemaphoreType.DMA,
        pltpu.SemaphoreType.DMA,
        pltpu.SemaphoreType.DMA,
        pltpu.SemaphoreType.DMA,
        pltpu.SemaphoreType.DMA,
        pltpu.SemaphoreType.DMA,
        pltpu.SemaphoreType.DMA,
        pltpu.SemaphoreType.DMA,
        pltpu.SemaphoreType.DMA,
        pltpu.SemaphoreType.DMA,
    ]
    if with_counts:
        out_type += [jax.ShapeDtypeStruct((N,), jnp.float32),
                     jax.ShapeDtypeStruct((N,), jnp.float32)]
        scratch += [pltpu.VMEM((128,), jnp.float32),
                    pltpu.VMEM_SHARED((N,), jnp.float32),
                    pltpu.SemaphoreType.DMA]

    def body(*refs):
        if with_counts:
            (tm, tu, es, ed, zz2, zz1,
             outm, outu, cntm, cntu,
             gring, sring, b0, b1, b2, acc,
             g0, g1, g2, s0, s1, s2, i0, i1, i2, i3,
             ones_v, cacc, csem) = refs
        else:
            (tm, tu, es, ed, zz2, zz1,
             outm, outu,
             gring, sring, b0, b1, b2, acc,
             g0, g1, g2, s0, s1, s2, i0, i1, i2, i3) = refs
        bufs = (b0, b1, b2)
        gsem = (g0, g1, g2)
        ssem = (s0, s1, s2)
        isem = (i0, i1, i2, i3)
        cid = lax.axis_index("c")
        sid = lax.axis_index("s")

        @pl.when(sid < 15)
        def _():
            pltpu.sync_copy(zz2, acc.at[pl.ds(sid * NT, NT)])

        @pl.when(sid == 15)
        def _():
            pltpu.sync_copy(zz2.at[pl.ds(0, NTL)], acc.at[pl.ds(15 * NT, NTL)])
        if with_counts:
            @pl.when(sid == 0)
            def _():
                pltpu.sync_copy(zz1, cacc)
            for k in range(8):
                ones_v[pl.ds(k * 16, 16)] = jnp.ones((16,), jnp.float32)

        plsc.subcore_barrier()

        def run_pass(table, g_hbm, s_hbm):
            row0 = sid * CPT

            def idx_slice(hbm, blk):
                return hbm.at[pl.ds(row0 + blk * BLK, BLK)]

            def ring_slice(ring, blk):
                return ring.at[pl.ds((blk % RING) * BLK, BLK)]

            def idx_fire(blk):
                sem = isem[blk % RING]
                pltpu.async_copy(idx_slice(g_hbm, blk), ring_slice(gring, blk), sem)
                pltpu.async_copy(idx_slice(s_hbm, blk), ring_slice(sring, blk), sem)

            def idx_wait(blk):
                sem = isem[blk % RING]
                pltpu.make_async_copy(idx_slice(g_hbm, blk),
                                      ring_slice(gring, blk), sem).wait()
                pltpu.make_async_copy(idx_slice(s_hbm, blk),
                                      ring_slice(sring, blk), sem).wait()

            def g_desc(c):
                slot = c % 3
                return pltpu.make_async_copy(
                    table.at[gring.at[c % (RING * BLK)]], bufs[slot], gsem[slot])

            def s_desc(c):
                slot = c % 3
                return pltpu.make_async_copy(
                    bufs[slot], acc.at[sring.at[c % (RING * BLK)]], ssem[slot])

            def c_desc(c):
                return pltpu.make_async_copy(
                    ones_v.at[pl.ds(0, CH)],
                    cacc.at[sring.at[c % (RING * BLK)]], csem)

            idx_fire(0)
            idx_fire(1)
            idx_waited = set()
            idx_wait(0)
            idx_waited.add(0)
            g_desc(0).start()
            if 1 // BLK not in idx_waited:
                idx_wait(1 // BLK)
                idx_waited.add(1 // BLK)
            g_desc(1).start()

            for c in range(CPT):
                if c % BLK == 0 and c // BLK + 2 < NBLK:
                    idx_fire(c // BLK + 2)
                g_desc(c).wait()
                s_desc(c).start(add=True)
                if with_counts:
                    if c >= 1:
                        c_desc(c - 1).wait()
                    c_desc(c).start(add=True)
                c2 = c + 2
                if c2 < CPT:
                    if c2 >= 3:
                        s_desc(c2 - 3).wait()
                    if c2 // BLK not in idx_waited:
                        idx_wait(c2 // BLK)
                        idx_waited.add(c2 // BLK)
                    g_desc(c2).start()

            for c in (CPT - 3, CPT - 2, CPT - 1):
                s_desc(c).wait()
            if with_counts:
                c_desc(CPT - 1).wait()

        @pl.when(cid == 0)
        def _():
            run_pass(tm, es, ed)

        @pl.when(cid == 1)
        def _():
            run_pass(tu, ed, es)

        plsc.subcore_barrier()

        def writeback(out, cnt_out, cacc_ref):
            @pl.when(sid < 15)
            def _():
                pltpu.sync_copy(acc.at[pl.ds(sid * NT, NT)],
                                out.at[pl.ds(sid * NT, NT)])

            @pl.when(sid == 15)
            def _():
                pltpu.sync_copy(acc.at[pl.ds(15 * NT, NTL)],
                                out.at[pl.ds(15 * NT, NTL)])
            if with_counts:
                @pl.when(sid == 0)
                def _():
                    pltpu.sync_copy(cacc_ref, cnt_out)

        @pl.when(cid == 0)
        def _():
            writeback(outm, cntm if with_counts else None,
                      cacc if with_counts else None)

        @pl.when(cid == 1)
        def _():
            writeback(outu, cntu if with_counts else None,
                      cacc if with_counts else None)

    k = pl.kernel(body, out_type=tuple(out_type), mesh=_MESH,
                  scratch_types=scratch)
    return k(table_m, table_u, esrc, edst, z2d, z1d)


def _movie_input(movie_x, w, b, emb):
    def body(mx, w_, b_, e_, o):
        o[...] = (jnp.dot(mx[...], w_[...], preferred_element_type=jnp.float32)
                  + b_[...] + e_[...])

    return pl.pallas_call(
        body,
        grid=(10,),
        in_specs=[pl.BlockSpec((1000, MF), lambda i: (i, 0)),
                  pl.BlockSpec((MF, H), lambda i: (0, 0)),
                  pl.BlockSpec((1, H), lambda i: (0, 0)),
                  pl.BlockSpec((1000, H), lambda i: (i, 0))],
        out_specs=pl.BlockSpec((1000, H), lambda i: (i, 0)),
        out_shape=jax.ShapeDtypeStruct((N, H), jnp.float32),
    )(movie_x, w, b, emb)


def _sage_linear(aggm, cbm, xm, wlm, blm, wrm,
                 aggu, cbu, xu, wlu, blu, wru, relu):
    def body(am, cm, xm_, wl1, bl1, wr1, au, cu, xu_, wl2, bl2, wr2, om, ou):
        mm = am[...] / jnp.maximum(cm[...], 1.0)
        hm = (jnp.dot(mm, wl1[...], preferred_element_type=jnp.float32)
              + bl1[...]
              + jnp.dot(xm_[...], wr1[...], preferred_element_type=jnp.float32))
        mu = au[...] / jnp.maximum(cu[...], 1.0)
        hu = (jnp.dot(mu, wl2[...], preferred_element_type=jnp.float32)
              + bl2[...]
              + jnp.dot(xu_[...], wr2[...], preferred_element_type=jnp.float32))
        if relu:
            hm = jnp.maximum(hm, 0.0)
            hu = jnp.maximum(hu, 0.0)
        om[...] = hm
        ou[...] = hu

    row = pl.BlockSpec((1000, H), lambda i: (i, 0))
    cnt = pl.BlockSpec((1000, 1), lambda i: (i, 0))
    wspec = pl.BlockSpec((H, H), lambda i: (0, 0))
    bspec = pl.BlockSpec((1, H), lambda i: (0, 0))
    return pl.pallas_call(
        body,
        grid=(10,),
        in_specs=[row, cnt, row, wspec, bspec, wspec,
                  row, cnt, row, wspec, bspec, wspec],
        out_specs=[row, row],
        out_shape=[jax.ShapeDtypeStruct((N, H), jnp.float32),
                   jax.ShapeDtypeStruct((N, H), jnp.float32)],
    )(aggm, cbm, xm, wlm, blm, wrm, aggu, cbu, xu, wlu, blu, wru)


def _edge_products(xu2, xm2, ls, ld):
    @functools.partial(
        pl.kernel,
        out_type=jax.ShapeDtypeStruct((LBL, H), jnp.float32),
        mesh=_MESH,
        scratch_types=[
            pltpu.VMEM((NCH, CCH), jnp.int32),
            pltpu.VMEM((NCH, CCH), jnp.int32),
            pltpu.VMEM((CCH, H), jnp.float32),
            pltpu.VMEM((CCH, H), jnp.float32),
            pltpu.VMEM((CCH, H), jnp.float32),
            pltpu.VMEM((CCH, H), jnp.float32),
            pltpu.SemaphoreType.DMA,
            pltpu.SemaphoreType.DMA,
            pltpu.SemaphoreType.DMA,
            pltpu.SemaphoreType.DMA,
        ],
    )
    def k(xu, xm, lsr, ldr, prod, iu, im, ub0, mb0, ub1, mb1,
          sema, semb, semo0, semo1):
        cid = lax.axis_index("c")
        sid = lax.axis_index("s")
        w = sid * 2 + cid
        base = w * EPW
        pltpu.sync_copy(lsr.at[pl.ds(w * NCH, NCH)], iu)
        pltpu.sync_copy(ldr.at[pl.ds(w * NCH, NCH)], im)

        def compute(ub, mb):
            def row_body(r, carry):
                for k in range(H // 16):
                    sl = pl.ds(k * 16, 16)
                    ub[r, sl] = ub[r, sl] * mb[r, sl]
                return carry

            lax.fori_loop(0, CCH, row_body, 0)

        def out_slice(c):
            return prod.at[pl.ds(base + c * CCH, CCH)]

        def gather(c, ub, mb, sem):
            pltpu.async_copy(xu.at[iu.at[c]], ub, sem)
            pltpu.async_copy(xm.at[im.at[c]], mb, sem)

        def gwait(c, ub, mb, sem):
            pltpu.make_async_copy(xu.at[iu.at[c]], ub, sem).wait()
            pltpu.make_async_copy(xm.at[im.at[c]], mb, sem).wait()

        gather(0, ub0, mb0, sema)

        def pair(c2, carry):
            c = c2 * 2
            gwait(c, ub0, mb0, sema)

            @pl.when(c >= 1)
            def _():
                pltpu.make_async_copy(ub1, out_slice(c - 1), semo1).wait()

            gather(c + 1, ub1, mb1, semb)
            compute(ub0, mb0)
            pltpu.async_copy(ub0, out_slice(c), semo0)
            gwait(c + 1, ub1, mb1, semb)

            @pl.when(c + 2 < NCH)
            def _():
                pltpu.make_async_copy(ub0, out_slice(c), semo0).wait()
                gather(c + 2, ub0, mb0, sema)

            compute(ub1, mb1)
            pltpu.async_copy(ub1, out_slice(c + 1), semo1)
            return carry

        lax.fori_loop(0, NCH // 2, pair, 0)
        pltpu.make_async_copy(ub0, out_slice(NCH - 2), semo0).wait()
        pltpu.make_async_copy(ub1, out_slice(NCH - 1), semo1).wait()

    return k(xu2, xm2, ls, ld)


def _row_sum(prod):
    def body(p, o):
        o[...] = jnp.sum(p[...], axis=-1)

    return pl.pallas_call(
        body,
        grid=(16,),
        in_specs=[pl.BlockSpec((LBL // 16, H), lambda i: (i, 0))],
        out_specs=pl.BlockSpec((LBL // 16,), lambda i: (i,)),
        out_shape=jax.ShapeDtypeStruct((LBL,), jnp.float32),
    )(prod)


def kernel(user_node_id, movie_node_id, movie_x, edge_index, edge_label_index,
           user_emb, movie_emb, movie_lin_w, movie_lin_b,
           c1m_wl, c1m_bl, c1m_wr, c1u_wl, c1u_bl, c1u_wr,
           c2m_wl, c2m_bl, c2m_wr, c2u_wl, c2u_bl, c2u_wr):
    x_user = user_emb
    x_movie = _movie_input(movie_x, movie_lin_w,
                           movie_lin_b.reshape(1, H), movie_emb)

    esrc = edge_index[0].reshape(ER, CH)
    edst = edge_index[1].reshape(ER, CH)
    z2d = jnp.zeros((NT, H), jnp.float32)
    z1d = jnp.zeros((N,), jnp.float32)

    agg1m, agg1u, cntm, cntu = _seg_sum(x_user, x_movie, esrc, edst,
                                        z2d, z1d, with_counts=True)
    cbm = cntm.reshape(N, 1)
    cbu = cntu.reshape(N, 1)
    xm1, xu1 = _sage_linear(agg1m, cbm, x_movie,
                            c1m_wl, c1m_bl.reshape(1, H), c1m_wr,
                            agg1u, cbu, x_user,
                            c1u_wl, c1u_bl.reshape(1, H), c1u_wr, relu=True)

    agg2m, agg2u = _seg_sum(xu1, xm1, esrc, edst, z2d, z1d,
                            with_counts=False)
    xm2, xu2 = _sage_linear(agg2m, cbm, xm1,
                            c2m_wl, c2m_bl.reshape(1, H), c2m_wr,
                            agg2u, cbu, xu1,
                            c2u_wl, c2u_bl.reshape(1, H), c2u_wr, relu=False)

    ls = edge_label_index[0].reshape(IR, CCH)
    ld = edge_label_index[1].reshape(IR, CCH)
    return _row_sum(_edge_products(xu2, xm2, ls, ld))


# speedup vs baseline: 1.0490x; 1.0270x over previous
import functools

import jax
import jax.numpy as jnp
from jax import lax
from jax.experimental import pallas as pl
from jax.experimental.pallas import tpu as pltpu
from jax.experimental.pallas import tpu_sc as plsc

N = 10000
H = 128
MF = 20
E = 320000
LBL = 65536
NSUB = 16

CH = 100
ER = E // CH
CPT = ER // NSUB
BLK = 8
RING = 4
NBLK = CPT // BLK
NT = 632
NTL = N - 15 * NT

CCH = 128
IR = LBL // CCH
NW = 2 * NSUB
EPW = LBL // NW
NCH = EPW // CCH

_MESH = plsc.VectorSubcoreMesh(core_axis_name="c", subcore_axis_name="s")


def _seg_sum(table_m, table_u, esrc, edst, z2d, z1d, with_counts):
    out_type = [jax.ShapeDtypeStruct((N, H), jnp.float32),
                jax.ShapeDtypeStruct((N, H), jnp.float32)]
    scratch = [
        pltpu.VMEM((RING * BLK, CH), jnp.int32),
        pltpu.VMEM((RING * BLK, CH), jnp.int32),
        pltpu.VMEM((CH, H), jnp.float32),
        pltpu.VMEM((CH, H), jnp.float32),
        pltpu.VMEM((CH, H), jnp.float32),
        pltpu.VMEM_SHARED((N, H), jnp.float32),
        pltpu.SemaphoreType.DMA,
        pltpu.SemaphoreType.DMA,
        pltpu.SemaphoreType.DMA,
        pltpu.SemaphoreType.DMA,
        pltpu.SemaphoreType.DMA,
        pltpu.SemaphoreType.DMA,
        pltpu.SemaphoreType.DMA,
        pltpu.SemaphoreType.DMA,
        pltpu.SemaphoreType.DMA,
        pltpu.SemaphoreType.DMA,
    ]
    if with_counts:
        out_type += [jax.ShapeDtypeStruct((N,), jnp.float32),
                     jax.ShapeDtypeStruct((N,), jnp.float32)]
        scratch += [pltpu.VMEM((128,), jnp.float32),
                    pltpu.VMEM_SHARED((N,), jnp.float32),
                    pltpu.SemaphoreType.DMA]

    def body(*refs):
        if with_counts:
            (tm, tu, es, ed, zz2, zz1,
             outm, outu, cntm, cntu,
             gring, sring, b0, b1, b2, acc,
             g0, g1, g2, s0, s1, s2, i0, i1, i2, i3,
             ones_v, cacc, csem) = refs
        else:
            (tm, tu, es, ed, zz2, zz1,
             outm, outu,
             gring, sring, b0, b1, b2, acc,
             g0, g1, g2, s0, s1, s2, i0, i1, i2, i3) = refs
        bufs = (b0, b1, b2)
        gsem = (g0, g1, g2)
        ssem = (s0, s1, s2)
        isem = (i0, i1, i2, i3)
        cid = lax.axis_index("c")
        sid = lax.axis_index("s")

        @pl.when(sid < 15)
        def _():
            pltpu.sync_copy(zz2, acc.at[pl.ds(sid * NT, NT)])

        @pl.when(sid == 15)
        def _():
            pltpu.sync_copy(zz2.at[pl.ds(0, NTL)], acc.at[pl.ds(15 * NT, NTL)])
        if with_counts:
            @pl.when(sid == 0)
            def _():
                pltpu.sync_copy(zz1, cacc)
            for k in range(8):
                ones_v[pl.ds(k * 16, 16)] = jnp.ones((16,), jnp.float32)

        plsc.subcore_barrier()

        def run_pass(table, g_hbm, s_hbm):
            row0 = sid * CPT

            def idx_slice(hbm, blk):
                return hbm.at[pl.ds(row0 + blk * BLK, BLK)]

            def ring_slice(ring, blk):
                return ring.at[pl.ds((blk % RING) * BLK, BLK)]

            def idx_fire(blk):
                sem = isem[blk % RING]
                pltpu.async_copy(idx_slice(g_hbm, blk), ring_slice(gring, blk), sem)
                pltpu.async_copy(idx_slice(s_hbm, blk), ring_slice(sring, blk), sem)

            def idx_wait(blk):
                sem = isem[blk % RING]
                pltpu.make_async_copy(idx_slice(g_hbm, blk),
                                      ring_slice(gring, blk), sem).wait()
                pltpu.make_async_copy(idx_slice(s_hbm, blk),
                                      ring_slice(sring, blk), sem).wait()

            def g_desc(c):
                slot = c % 3
                return pltpu.make_async_copy(
                    table.at[gring.at[c % (RING * BLK)]], bufs[slot], gsem[slot])

            def s_desc(c):
                slot = c % 3
                return pltpu.make_async_copy(
                    bufs[slot], acc.at[sring.at[c % (RING * BLK)]], ssem[slot])

            def c_desc(c):
                return pltpu.make_async_copy(
                    ones_v.at[pl.ds(0, CH)],
                    cacc.at[sring.at[c % (RING * BLK)]], csem)

            idx_fire(0)
            idx_fire(1)
            idx_waited = set()
            idx_wait(0)
            idx_waited.add(0)
            g_desc(0).start()
            if 1 // BLK not in idx_waited:
                idx_wait(1 // BLK)
                idx_waited.add(1 // BLK)
            g_desc(1).start()

            for c in range(CPT):
                if c % BLK == 0 and c // BLK + 2 < NBLK:
                    idx_fire(c // BLK + 2)
                g_desc(c).wait()
                s_desc(c).start(add=True)
                if with_counts:
                    if c >= 1:
                        c_desc(c - 1).wait()
                    c_desc(c).start(add=True)
                c2 = c + 2
                if c2 < CPT:
                    if c2 >= 3:
                        s_desc(c2 - 3).wait()
                    if c2 // BLK not in idx_waited:
                        idx_wait(c2 // BLK)
                        idx_waited.add(c2 // BLK)
                    g_desc(c2).start()

            for c in (CPT - 3, CPT - 2, CPT - 1):
                s_desc(c).wait()
            if with_counts:
                c_desc(CPT - 1).wait()

        @pl.when(cid == 0)
        def _():
            run_pass(tm, es, ed)

        @pl.when(cid == 1)
        def _():
            run_pass(tu, ed, es)

        plsc.subcore_barrier()

        def writeback(out, cnt_out, cacc_ref):
            @pl.when(sid < 15)
            def _():
                pltpu.sync_copy(acc.at[pl.ds(sid * NT, NT)],
                                out.at[pl.ds(sid * NT, NT)])

            @pl.when(sid == 15)
            def _():
                pltpu.sync_copy(acc.at[pl.ds(15 * NT, NTL)],
                                out.at[pl.ds(15 * NT, NTL)])
            if with_counts:
                @pl.when(sid == 0)
                def _():
                    pltpu.sync_copy(cacc_ref, cnt_out)

        @pl.when(cid == 0)
        def _():
            writeback(outm, cntm if with_counts else None,
                      cacc if with_counts else None)

        @pl.when(cid == 1)
        def _():
            writeback(outu, cntu if with_counts else None,
                      cacc if with_counts else None)

    k = pl.kernel(body, out_type=tuple(out_type), mesh=_MESH,
                  scratch_types=scratch)
    return k(table_m, table_u, esrc, edst, z2d, z1d)


def _movie_input(movie_x, w, b, emb):
    def body(mx, w_, b_, e_, o):
        o[...] = (jnp.dot(mx[...], w_[...], preferred_element_type=jnp.float32)
                  + b_[...] + e_[...])

    return pl.pallas_call(
        body,
        grid=(10,),
        in_specs=[pl.BlockSpec((1000, MF), lambda i: (i, 0)),
                  pl.BlockSpec((MF, H), lambda i: (0, 0)),
                  pl.BlockSpec((1, H), lambda i: (0, 0)),
                  pl.BlockSpec((1000, H), lambda i: (i, 0))],
        out_specs=pl.BlockSpec((1000, H), lambda i: (i, 0)),
        out_shape=jax.ShapeDtypeStruct((N, H), jnp.float32),
    )(movie_x, w, b, emb)


def _sage_linear(aggm, cbm, xm, wlm, blm, wrm,
                 aggu, cbu, xu, wlu, blu, wru, relu):
    def body(am, cm, xm_, wl1, bl1, wr1, au, cu, xu_, wl2, bl2, wr2, om, ou):
        mm = am[...] / jnp.maximum(cm[...], 1.0)
        hm = (jnp.dot(mm, wl1[...], preferred_element_type=jnp.float32)
              + bl1[...]
              + jnp.dot(xm_[...], wr1[...], preferred_element_type=jnp.float32))
        mu = au[...] / jnp.maximum(cu[...], 1.0)
        hu = (jnp.dot(mu, wl2[...], preferred_element_type=jnp.float32)
              + bl2[...]
              + jnp.dot(xu_[...], wr2[...], preferred_element_type=jnp.float32))
        if relu:
            hm = jnp.maximum(hm, 0.0)
            hu = jnp.maximum(hu, 0.0)
        om[...] = hm
        ou[...] = hu

    row = pl.BlockSpec((1000, H), lambda i: (i, 0))
    cnt = pl.BlockSpec((1000, 1), lambda i: (i, 0))
    wspec = pl.BlockSpec((H, H), lambda i: (0, 0))
    bspec = pl.BlockSpec((1, H), lambda i: (0, 0))
    return pl.pallas_call(
        body,
        grid=(10,),
        in_specs=[row, cnt, row, wspec, bspec, wspec,
                  row, cnt, row, wspec, bspec, wspec],
        out_specs=[row, row],
        out_shape=[jax.ShapeDtypeStruct((N, H), jnp.float32),
                   jax.ShapeDtypeStruct((N, H), jnp.float32)],
    )(aggm, cbm, xm, wlm, blm, wrm, aggu, cbu, xu, wlu, blu, wru)


def _classifier(xu2, xm2, ls, ld):
    @functools.partial(
        pl.kernel,
        out_type=jax.ShapeDtypeStruct((LBL,), jnp.float32),
        mesh=_MESH,
        scratch_types=[
            pltpu.VMEM((NCH, CCH), jnp.int32),
            pltpu.VMEM((NCH, CCH), jnp.int32),
            pltpu.VMEM((CCH, H), jnp.float32),
            pltpu.VMEM((CCH, H), jnp.float32),
            pltpu.VMEM((CCH, H), jnp.float32),
            pltpu.VMEM((CCH, H), jnp.float32),
            pltpu.VMEM((EPW,), jnp.float32),
            pltpu.SemaphoreType.DMA,
            pltpu.SemaphoreType.DMA,
        ],
    )
    def k(xu, xm, lsr, ldr, pred, iu, im, ub0, mb0, ub1, mb1, obuf,
          sema, semb):
        cid = lax.axis_index("c")
        sid = lax.axis_index("s")
        w = sid * 2 + cid
        pltpu.sync_copy(lsr.at[pl.ds(w * NCH, NCH)], iu)
        pltpu.sync_copy(ldr.at[pl.ds(w * NCH, NCH)], im)
        lanes = lax.iota(jnp.int32, 16)

        def hsum(s):
            for kk in (8, 4, 2, 1):
                perm = jnp.bitwise_xor(lanes, kk)
                s = s + s.at[perm].get(mode="promise_in_bounds")
            return s

        def compute(ub, mb, c):
            def grp(g, carry):
                acc = jnp.zeros((16,), jnp.float32)
                for j in range(16):
                    r = g * 16 + j
                    s = ub[r, pl.ds(0, 16)] * mb[r, pl.ds(0, 16)]
                    for kk in range(1, H // 16):
                        sl = pl.ds(kk * 16, 16)
                        s = s + ub[r, sl] * mb[r, sl]
                    acc = jnp.where(lanes == j, hsum(s), acc)
                obuf[pl.ds(c * CCH + g * 16, 16)] = acc
                return carry

            lax.fori_loop(0, CCH // 16, grp, 0)

        def gather(c, ub, mb, sem):
            pltpu.async_copy(xu.at[iu.at[c]], ub, sem)
            pltpu.async_copy(xm.at[im.at[c]], mb, sem)

        def gwait(c, ub, mb, sem):
            pltpu.make_async_copy(xu.at[iu.at[c]], ub, sem).wait()
            pltpu.make_async_copy(xm.at[im.at[c]], mb, sem).wait()

        gather(0, ub0, mb0, sema)

        def pair(c2, carry):
            c = c2 * 2
            gwait(c, ub0, mb0, sema)
            gather(c + 1, ub1, mb1, semb)
            compute(ub0, mb0, c)
            gwait(c + 1, ub1, mb1, semb)

            @pl.when(c + 2 < NCH)
            def _():
                gather(c + 2, ub0, mb0, sema)

            compute(ub1, mb1, c + 1)
            return carry

        lax.fori_loop(0, NCH // 2, pair, 0)
        pltpu.sync_copy(obuf, pred.at[pl.ds(w * EPW, EPW)])

    return k(xu2, xm2, ls, ld)


def kernel(user_node_id, movie_node_id, movie_x, edge_index, edge_label_index,
           user_emb, movie_emb, movie_lin_w, movie_lin_b,
           c1m_wl, c1m_bl, c1m_wr, c1u_wl, c1u_bl, c1u_wr,
           c2m_wl, c2m_bl, c2m_wr, c2u_wl, c2u_bl, c2u_wr):
    x_user = user_emb
    x_movie = _movie_input(movie_x, movie_lin_w,
                           movie_lin_b.reshape(1, H), movie_emb)

    esrc = edge_index[0].reshape(ER, CH)
    edst = edge_index[1].reshape(ER, CH)
    z2d = jnp.zeros((NT, H), jnp.float32)
    z1d = jnp.zeros((N,), jnp.float32)

    agg1m, agg1u, cntm, cntu = _seg_sum(x_user, x_movie, esrc, edst,
                                        z2d, z1d, with_counts=True)
    cbm = cntm.reshape(N, 1)
    cbu = cntu.reshape(N, 1)
    xm1, xu1 = _sage_linear(agg1m, cbm, x_movie,
                            c1m_wl, c1m_bl.reshape(1, H), c1m_wr,
                            agg1u, cbu, x_user,
                            c1u_wl, c1u_bl.reshape(1, H), c1u_wr, relu=True)

    agg2m, agg2u = _seg_sum(xu1, xm1, esrc, edst, z2d, z1d,
                            with_counts=False)
    xm2, xu2 = _sage_linear(agg2m, cbm, xm1,
                            c2m_wl, c2m_bl.reshape(1, H), c2m_wr,
                            agg2u, cbu, xu1,
                            c2u_wl, c2u_bl.reshape(1, H), c2u_wr, relu=False)

    ls = edge_label_index[0].reshape(IR, CCH)
    ld = edge_label_index[1].reshape(IR, CCH)
    return _classifier(xu2, xm2, ls, ld)


# speedup vs baseline: 1.0898x; 1.0390x over previous
import functools

import jax
import jax.numpy as jnp
from jax import lax
from jax.experimental import pallas as pl
from jax.experimental.pallas import tpu as pltpu
from jax.experimental.pallas import tpu_sc as plsc

N = 10000
H = 128
MF = 20
E = 320000
LBL = 65536
NSUB = 16

CH = 100
ER = E // CH
CPT = ER // NSUB
BLK = 8
RING = 4
NBLK = CPT // BLK
NT = 632
NTL = N - 15 * NT

CCH = 128
IR = LBL // CCH
NW = 2 * NSUB
EPW = LBL // NW
NCH = EPW // CCH

_MESH = plsc.VectorSubcoreMesh(core_axis_name="c", subcore_axis_name="s")


def _seg_sum(table_m, table_u, eidx, z2d, z1d, with_counts):
    out_type = [jax.ShapeDtypeStruct((N, H), jnp.float32),
                jax.ShapeDtypeStruct((N, H), jnp.float32)]
    scratch = [
        pltpu.VMEM((RING * BLK, CH), jnp.int32),
        pltpu.VMEM((RING * BLK, CH), jnp.int32),
        pltpu.VMEM((CH, H), jnp.float32),
        pltpu.VMEM((CH, H), jnp.float32),
        pltpu.VMEM((CH, H), jnp.float32),
        pltpu.VMEM_SHARED((N, H), jnp.float32),
        pltpu.SemaphoreType.DMA,
        pltpu.SemaphoreType.DMA,
        pltpu.SemaphoreType.DMA,
        pltpu.SemaphoreType.DMA,
        pltpu.SemaphoreType.DMA,
        pltpu.SemaphoreType.DMA,
        pltpu.SemaphoreType.DMA,
        pltpu.SemaphoreType.DMA,
        pltpu.SemaphoreType.DMA,
        pltpu.SemaphoreType.DMA,
    ]
    if with_counts:
        out_type += [jax.ShapeDtypeStruct((N,), jnp.float32),
                     jax.ShapeDtypeStruct((N,), jnp.float32)]
        scratch += [pltpu.VMEM((128,), jnp.float32),
                    pltpu.VMEM_SHARED((N,), jnp.float32),
                    pltpu.SemaphoreType.DMA]

    def body(*refs):
        if with_counts:
            (tm, tu, e3, zz2, zz1,
             outm, outu, cntm, cntu,
             gring, sring, b0, b1, b2, acc,
             g0, g1, g2, s0, s1, s2, i0, i1, i2, i3,
             ones_v, cacc, csem) = refs
        else:
            (tm, tu, e3, zz2, zz1,
             outm, outu,
             gring, sring, b0, b1, b2, acc,
             g0, g1, g2, s0, s1, s2, i0, i1, i2, i3) = refs
        bufs = (b0, b1, b2)
        gsem = (g0, g1, g2)
        ssem = (s0, s1, s2)
        isem = (i0, i1, i2, i3)
        cid = lax.axis_index("c")
        sid = lax.axis_index("s")

        @pl.when(sid < 15)
        def _():
            pltpu.sync_copy(zz2, acc.at[pl.ds(sid * NT, NT)])

        @pl.when(sid == 15)
        def _():
            pltpu.sync_copy(zz2.at[pl.ds(0, NTL)], acc.at[pl.ds(15 * NT, NTL)])
        if with_counts:
            @pl.when(sid == 0)
            def _():
                pltpu.sync_copy(zz1, cacc)
            for k in range(8):
                ones_v[pl.ds(k * 16, 16)] = jnp.ones((16,), jnp.float32)

        plsc.subcore_barrier()

        def run_pass(table, gd, sd):
            row0 = sid * CPT

            def idx_slice(d, blk):
                return e3.at[d, pl.ds(row0 + blk * BLK, BLK)]

            def ring_slice(ring, blk):
                return ring.at[pl.ds((blk % RING) * BLK, BLK)]

            def idx_fire(blk):
                sem = isem[blk % RING]
                pltpu.async_copy(idx_slice(gd, blk), ring_slice(gring, blk), sem)
                pltpu.async_copy(idx_slice(sd, blk), ring_slice(sring, blk), sem)

            def idx_wait(blk):
                sem = isem[blk % RING]
                pltpu.make_async_copy(idx_slice(gd, blk),
                                      ring_slice(gring, blk), sem).wait()
                pltpu.make_async_copy(idx_slice(sd, blk),
                                      ring_slice(sring, blk), sem).wait()

            def g_desc(c):
                slot = c % 3
                return pltpu.make_async_copy(
                    table.at[gring.at[c % (RING * BLK)]], bufs[slot], gsem[slot])

            def s_desc(c):
                slot = c % 3
                return pltpu.make_async_copy(
                    bufs[slot], acc.at[sring.at[c % (RING * BLK)]], ssem[slot])

            def c_desc(c):
                return pltpu.make_async_copy(
                    ones_v.at[pl.ds(0, CH)],
                    cacc.at[sring.at[c % (RING * BLK)]], csem)

            idx_fire(0)
            idx_fire(1)
            idx_waited = set()
            idx_wait(0)
            idx_waited.add(0)
            g_desc(0).start()
            if 1 // BLK not in idx_waited:
                idx_wait(1 // BLK)
                idx_waited.add(1 // BLK)
            g_desc(1).start()

            for c in range(CPT):
                if c % BLK == 0 and c // BLK + 2 < NBLK:
                    idx_fire(c // BLK + 2)
                g_desc(c).wait()
                s_desc(c).start(add=True)
                if with_counts:
                    if c >= 1:
                        c_desc(c - 1).wait()
                    c_desc(c).start(add=True)
                c2 = c + 2
                if c2 < CPT:
                    if c2 >= 3:
                        s_desc(c2 - 3).wait()
                    if c2 // BLK not in idx_waited:
                        idx_wait(c2 // BLK)
                        idx_waited.add(c2 // BLK)
                    g_desc(c2).start()

            for c in (CPT - 3, CPT - 2, CPT - 1):
                s_desc(c).wait()
            if with_counts:
                c_desc(CPT - 1).wait()

        @pl.when(cid == 0)
        def _():
            run_pass(tm, 0, 1)

        @pl.when(cid == 1)
        def _():
            run_pass(tu, 1, 0)

        plsc.subcore_barrier()

        def writeback(out, cnt_out, cacc_ref):
            @pl.when(sid < 15)
            def _():
                pltpu.sync_copy(acc.at[pl.ds(sid * NT, NT)],
                                out.at[pl.ds(sid * NT, NT)])

            @pl.when(sid == 15)
            def _():
                pltpu.sync_copy(acc.at[pl.ds(15 * NT, NTL)],
                                out.at[pl.ds(15 * NT, NTL)])
            if with_counts:
                @pl.when(sid == 0)
                def _():
                    pltpu.sync_copy(cacc_ref, cnt_out)

        @pl.when(cid == 0)
        def _():
            writeback(outm, cntm if with_counts else None,
                      cacc if with_counts else None)

        @pl.when(cid == 1)
        def _():
            writeback(outu, cntu if with_counts else None,
                      cacc if with_counts else None)

    k = pl.kernel(body, out_type=tuple(out_type), mesh=_MESH,
                  scratch_types=scratch)
    return k(table_m, table_u, eidx, z2d, z1d)


def _movie_input(movie_x, w, b, emb):
    def body(mx, w_, b_, e_, o):
        o[...] = (jnp.dot(mx[...], w_[...], preferred_element_type=jnp.float32)
                  + b_[...] + e_[...])

    return pl.pallas_call(
        body,
        grid=(5,),
        in_specs=[pl.BlockSpec((2000, MF), lambda i: (i, 0)),
                  pl.BlockSpec((MF, H), lambda i: (0, 0)),
                  pl.BlockSpec((1, H), lambda i: (0, 0)),
                  pl.BlockSpec((2000, H), lambda i: (i, 0))],
        out_specs=pl.BlockSpec((2000, H), lambda i: (i, 0)),
        out_shape=jax.ShapeDtypeStruct((N, H), jnp.float32),
    )(movie_x, w, b, emb)


def _sage_linear(aggm, cbm, xm, wlm, blm, wrm,
                 aggu, cbu, xu, wlu, blu, wru, relu):
    def body(am, cm, xm_, wl1, bl1, wr1, au, cu, xu_, wl2, bl2, wr2, om, ou):
        mm = am[...] / jnp.maximum(cm[...], 1.0)
        hm = (jnp.dot(mm, wl1[...], preferred_element_type=jnp.float32)
              + bl1[...]
              + jnp.dot(xm_[...], wr1[...], preferred_element_type=jnp.float32))
        mu = au[...] / jnp.maximum(cu[...], 1.0)
        hu = (jnp.dot(mu, wl2[...], preferred_element_type=jnp.float32)
              + bl2[...]
              + jnp.dot(xu_[...], wr2[...], preferred_element_type=jnp.float32))
        if relu:
            hm = jnp.maximum(hm, 0.0)
            hu = jnp.maximum(hu, 0.0)
        om[...] = hm
        ou[...] = hu

    row = pl.BlockSpec((2000, H), lambda i: (i, 0))
    cnt = pl.BlockSpec((2000, 1), lambda i: (i, 0))
    wspec = pl.BlockSpec((H, H), lambda i: (0, 0))
    bspec = pl.BlockSpec((1, H), lambda i: (0, 0))
    return pl.pallas_call(
        body,
        grid=(5,),
        in_specs=[row, cnt, row, wspec, bspec, wspec,
                  row, cnt, row, wspec, bspec, wspec],
        out_specs=[row, row],
        out_shape=[jax.ShapeDtypeStruct((N, H), jnp.float32),
                   jax.ShapeDtypeStruct((N, H), jnp.float32)],
    )(aggm, cbm, xm, wlm, blm, wrm, aggu, cbu, xu, wlu, blu, wru)


def _classifier(xu2, xm2, lidx):
    @functools.partial(
        pl.kernel,
        out_type=jax.ShapeDtypeStruct((LBL,), jnp.float32),
        mesh=_MESH,
        scratch_types=[
            pltpu.VMEM((NCH, CCH), jnp.int32),
            pltpu.VMEM((NCH, CCH), jnp.int32),
            pltpu.VMEM((CCH, H), jnp.float32),
            pltpu.VMEM((CCH, H), jnp.float32),
            pltpu.VMEM((CCH, H), jnp.float32),
            pltpu.VMEM((CCH, H), jnp.float32),
            pltpu.VMEM((EPW,), jnp.float32),
            pltpu.SemaphoreType.DMA,
            pltpu.SemaphoreType.DMA,
        ],
    )
    def k(xu, xm, l3, pred, iu, im, ub0, mb0, ub1, mb1, obuf,
          sema, semb):
        cid = lax.axis_index("c")
        sid = lax.axis_index("s")
        w = sid * 2 + cid
        pltpu.sync_copy(l3.at[0, pl.ds(w * NCH, NCH)], iu)
        pltpu.sync_copy(l3.at[1, pl.ds(w * NCH, NCH)], im)
        lanes = lax.iota(jnp.int32, 16)

        def hsum(s):
            for kk in (8, 4, 2, 1):
                perm = jnp.bitwise_xor(lanes, kk)
                s = s + s.at[perm].get(mode="promise_in_bounds")
            return s

        def compute(ub, mb, c):
            def grp(g, carry):
                acc = jnp.zeros((16,), jnp.float32)
                for j in range(16):
                    r = g * 16 + j
                    s = ub[r, pl.ds(0, 16)] * mb[r, pl.ds(0, 16)]
                    for kk in range(1, H // 16):
                        sl = pl.ds(kk * 16, 16)
                        s = s + ub[r, sl] * mb[r, sl]
                    acc = jnp.where(lanes == j, hsum(s), acc)
                obuf[pl.ds(c * CCH + g * 16, 16)] = acc
                return carry

            lax.fori_loop(0, CCH // 16, grp, 0)

        def gather(c, ub, mb, sem):
            pltpu.async_copy(xu.at[iu.at[c]], ub, sem)
            pltpu.async_copy(xm.at[im.at[c]], mb, sem)

        def gwait(c, ub, mb, sem):
            pltpu.make_async_copy(xu.at[iu.at[c]], ub, sem).wait()
            pltpu.make_async_copy(xm.at[im.at[c]], mb, sem).wait()

        gather(0, ub0, mb0, sema)

        def pair(c2, carry):
            c = c2 * 2
            gwait(c, ub0, mb0, sema)
            gather(c + 1, ub1, mb1, semb)
            compute(ub0, mb0, c)
            gwait(c + 1, ub1, mb1, semb)

            @pl.when(c + 2 < NCH)
            def _():
                gather(c + 2, ub0, mb0, sema)

            compute(ub1, mb1, c + 1)
            return carry

        lax.fori_loop(0, NCH // 2, pair, 0)
        pltpu.sync_copy(obuf, pred.at[pl.ds(w * EPW, EPW)])

    return k(xu2, xm2, lidx)


def kernel(user_node_id, movie_node_id, movie_x, edge_index, edge_label_index,
           user_emb, movie_emb, movie_lin_w, movie_lin_b,
           c1m_wl, c1m_bl, c1m_wr, c1u_wl, c1u_bl, c1u_wr,
           c2m_wl, c2m_bl, c2m_wr, c2u_wl, c2u_bl, c2u_wr):
    x_user = user_emb
    x_movie = _movie_input(movie_x, movie_lin_w,
                           movie_lin_b.reshape(1, H), movie_emb)

    eidx = edge_index.reshape(2, ER, CH)
    z2d = jnp.zeros((NT, H), jnp.float32)
    z1d = jnp.zeros((N,), jnp.float32)

    agg1m, agg1u, cntm, cntu = _seg_sum(x_user, x_movie, eidx,
                                        z2d, z1d, with_counts=True)
    cbm = cntm.reshape(N, 1)
    cbu = cntu.reshape(N, 1)
    xm1, xu1 = _sage_linear(agg1m, cbm, x_movie,
                            c1m_wl, c1m_bl.reshape(1, H), c1m_wr,
                            agg1u, cbu, x_user,
                            c1u_wl, c1u_bl.reshape(1, H), c1u_wr, relu=True)

    agg2m, agg2u = _seg_sum(xu1, xm1, eidx, z2d, z1d,
                            with_counts=False)
    xm2, xu2 = _sage_linear(agg2m, cbm, xm1,
                            c2m_wl, c2m_bl.reshape(1, H), c2m_wr,
                            agg2u, cbu, xu1,
                            c2u_wl, c2u_bl.reshape(1, H), c2u_wr, relu=False)

    lidx = edge_label_index.reshape(2, IR, CCH)
    return _classifier(xu2, xm2, lidx)
